# Initial kernel scaffold; baseline (speedup 1.0000x reference)
import jax
import jax.numpy as jnp
from jax.experimental import pallas as pl


def kernel(x, gate_w, w1, w2, w3):
    raise NotImplementedError("write your pallas kernel here")


# speedup vs baseline: 4.0047x; 4.0047x over previous
import functools

import jax
import jax.numpy as jnp
from jax import lax
from jax.experimental import pallas as pl
from jax.experimental.pallas import tpu as pltpu
from jax.experimental.pallas import tpu_sc as plsc

DIM_ = 1024
HID_ = 2816
E_ = 8
K_ = 2

BM = 512
BH = 1408
G_GATHER = 64
G_TOK = 32
NC_SC = 2
NS_SC = 16
NW_SC = NC_SC * NS_SC


def _routing_metadata(sel_flat, m_rows, num_tiles, num_units):
  sizes = jnp.bincount(sel_flat, length=E_).astype(jnp.int32)
  offs = jnp.concatenate(
      [jnp.zeros((1,), jnp.int32), jnp.cumsum(sizes).astype(jnp.int32)])
  t_lo = offs[:-1] // BM
  t_hi = (offs[1:] + BM - 1) // BM
  ntiles = jnp.where(sizes > 0, t_hi - t_lo, 0)
  starts = jnp.concatenate(
      [jnp.zeros((1,), jnp.int32), jnp.cumsum(ntiles).astype(jnp.int32)])
  total = starts[-1]
  j = jnp.arange(num_units, dtype=jnp.int32)
  e_of = jnp.sum(j[:, None] >= starts[None, 1:], axis=1).astype(jnp.int32)
  e_of = jnp.minimum(e_of, E_ - 1)
  t_of = t_lo[e_of] + j - starts[e_of]
  valid = j < total
  last = total - 1
  t_of = jnp.where(valid, t_of, jnp.take(t_of, last))
  e_of = jnp.where(valid, e_of, jnp.take(e_of, last))
  prev_t = jnp.roll(t_of, 1)
  isfirst = valid & ((j == 0) | (t_of != prev_t))
  flags = valid.astype(jnp.int32) | (isfirst.astype(jnp.int32) << 1)
  return t_of, e_of, flags, offs


def _gmm_body(t_ref, e_ref, f_ref, o_ref, x_ref, s_ref, w1_ref, w3_ref,
              w2_ref, out_ref):
  i = pl.program_id(0)
  h = pl.program_id(1)
  fl = f_ref[i]
  valid = (fl & 1) == 1
  first = (fl & 2) == 2
  e = e_ref[i]
  t = t_ref[i]
  start = o_ref[e]
  end = o_ref[e + 1]

  @pl.when(valid)
  def _():
    rows = t * BM + lax.broadcasted_iota(jnp.int32, (BM, 1), 0)
    mask = (rows >= start) & (rows < end)
    s = jnp.where(mask, s_ref[:, 0:1], 0.0)
    xm = x_ref[...] * s
    h1 = jnp.dot(xm, w1_ref[0], preferred_element_type=jnp.float32)
    h3 = jnp.dot(xm, w3_ref[0], preferred_element_type=jnp.float32)
    g = h1 * jax.nn.sigmoid(h1) * h3
    o = jnp.dot(g, w2_ref[0], preferred_element_type=jnp.float32)
    init = first & (h == 0)

    @pl.when(init)
    def _():
      out_ref[...] = o

    @pl.when(jnp.logical_not(init))
    def _():
      out_ref[...] += o


def _grouped_mlp(x_sorted, scores_bcast, w1, w2, w3, t_of, e_of, flags, offs,
                 m_rows, num_units, interpret=False):
  nh = HID_ // BH
  grid_spec = pltpu.PrefetchScalarGridSpec(
      num_scalar_prefetch=4,
      grid=(num_units, nh),
      in_specs=[
          pl.BlockSpec((BM, DIM_), lambda i, h, t, e, f, o: (t[i], 0)),
          pl.BlockSpec((BM, 128), lambda i, h, t, e, f, o: (t[i], 0)),
          pl.BlockSpec((1, DIM_, BH), lambda i, h, t, e, f, o: (e[i], 0, h)),
          pl.BlockSpec((1, DIM_, BH), lambda i, h, t, e, f, o: (e[i], 0, h)),
          pl.BlockSpec((1, BH, DIM_), lambda i, h, t, e, f, o: (e[i], h, 0)),
      ],
      out_specs=pl.BlockSpec((BM, DIM_), lambda i, h, t, e, f, o: (t[i], 0)),
  )
  return pl.pallas_call(
      _gmm_body,
      grid_spec=grid_spec,
      out_shape=jax.ShapeDtypeStruct((m_rows, DIM_), jnp.float32),
      compiler_params=pltpu.CompilerParams(
          dimension_semantics=("arbitrary", "arbitrary")),
      interpret=interpret,
  )(t_of, e_of, flags, offs, x_sorted, scores_bcast, w1, w3, w2)


def _sc_gather(gather_idx, xf, m_rows):
  rows_per_w = m_rows // NW_SC
  mesh = plsc.VectorSubcoreMesh(
      core_axis_name="c", subcore_axis_name="s")

  @functools.partial(
      pl.kernel,
      out_type=jax.ShapeDtypeStruct((m_rows, DIM_), jnp.float32),
      mesh=mesh,
      scratch_types=[
          pltpu.VMEM((G_GATHER,), jnp.int32),
          pltpu.VMEM((G_GATHER, DIM_), jnp.float32),
          pltpu.SemaphoreType.DMA,
      ],
  )
  def k(idx_hbm, x_hbm, out_hbm, idx_v, rows_v, sem):
    wid = lax.axis_index("s") * NC_SC + lax.axis_index("c")
    base = wid * rows_per_w
    for c in range(rows_per_w // G_GATHER):
      off = base + c * G_GATHER
      pltpu.sync_copy(idx_hbm.at[pl.ds(off, G_GATHER)], idx_v)
      pltpu.async_copy(x_hbm.at[idx_v], rows_v, sem).wait()
      pltpu.sync_copy(rows_v, out_hbm.at[pl.ds(off, G_GATHER)])

  return k(gather_idx, xf)


def _sc_combine(inv_idx, routed_out, n_tok):
  tok_per_w = n_tok // NW_SC
  mesh = plsc.VectorSubcoreMesh(
      core_axis_name="c", subcore_axis_name="s")

  @functools.partial(
      pl.kernel,
      out_type=jax.ShapeDtypeStruct((n_tok, DIM_), jnp.float32),
      mesh=mesh,
      scratch_types=[
          pltpu.VMEM((2 * G_TOK,), jnp.int32),
          pltpu.VMEM((2 * G_TOK, DIM_), jnp.float32),
          pltpu.VMEM((G_TOK, DIM_), jnp.float32),
          pltpu.SemaphoreType.DMA,
      ],
  )
  def k(inv_hbm, ro_hbm, out_hbm, idx_v, rows_v, o_v, sem):
    wid = lax.axis_index("s") * NC_SC + lax.axis_index("c")
    tbase = wid * tok_per_w
    for c in range(tok_per_w // G_TOK):
      t0 = tbase + c * G_TOK
      pltpu.sync_copy(inv_hbm.at[pl.ds(2 * t0, 2 * G_TOK)], idx_v)
      pltpu.async_copy(ro_hbm.at[idx_v], rows_v, sem).wait()

      def tok_body(i, _):
        def sl_body(cc, __):
          a = rows_v[2 * i, pl.ds(cc * 16, 16)]
          b = rows_v[2 * i + 1, pl.ds(cc * 16, 16)]
          o_v[i, pl.ds(cc * 16, 16)] = a + b
          return 0

        return lax.fori_loop(0, DIM_ // 16, sl_body, 0)

      lax.fori_loop(0, G_TOK, tok_body, 0)
      pltpu.sync_copy(o_v, out_hbm.at[pl.ds(t0, G_TOK)])

  return k(inv_idx, routed_out)


def kernel(x, gate_w, w1, w2, w3):
  bs, slen, dim = x.shape
  n_tok = bs * slen
  m_rows = n_tok * K_
  num_tiles = m_rows // BM
  num_units = num_tiles + E_ - 1
  xf = x.reshape(n_tok, dim)

  logits = xf @ gate_w.T
  scores = jax.nn.softmax(logits.astype(jnp.float32), axis=1)
  top_scores, selected = lax.top_k(scores, K_)
  sel_flat = selected.reshape(-1).astype(jnp.int32)

  sort_idx = jnp.argsort(sel_flat, stable=True).astype(jnp.int32)
  inv_idx = jnp.zeros((m_rows,), jnp.int32).at[sort_idx].set(
      jnp.arange(m_rows, dtype=jnp.int32))
  gather_idx = sort_idx // K_
  scores_sorted = top_scores.reshape(-1)[sort_idx]
  t_of, e_of, flags, offs = _routing_metadata(
      sel_flat, m_rows, num_tiles, num_units)

  x_sorted = _sc_gather(gather_idx, xf, m_rows)

  scores_bcast = jnp.broadcast_to(scores_sorted[:, None], (m_rows, 128))
  routed_out = _grouped_mlp(x_sorted, scores_bcast, w1, w2, w3, t_of, e_of,
                            flags, offs, m_rows, num_units)

  out = _sc_combine(inv_idx, routed_out, n_tok)
  return out.reshape(bs, slen, dim)
